# Initial kernel scaffold; baseline (speedup 1.0000x reference)
import jax
import jax.numpy as jnp
from jax.experimental import pallas as pl


def kernel(x, y, embedding):
    raise NotImplementedError("write your pallas kernel here")


# speedup vs baseline: 1.9244x; 1.9244x over previous
import functools

import jax
import jax.numpy as jnp
from jax import lax
from jax.experimental import pallas as pl
from jax.experimental.pallas import tpu as pltpu
from jax.experimental.pallas import tpu_sc as plsc

N = 16384
D = 128
CHUNK = 128
NC = 2
NS = 16
NW = NC * NS
ROWS_PER_W = N // NW
CHUNKS_PER_W = ROWS_PER_W // CHUNK
NIDX_ROWS = N // CHUNK

_mesh = plsc.VectorSubcoreMesh(core_axis_name="c", subcore_axis_name="s")


@functools.partial(
    pl.kernel,
    mesh=_mesh,
    out_type=jax.ShapeDtypeStruct((N, 2 * D), jnp.float32),
    scratch_types=[
        pltpu.VMEM((CHUNK,), jnp.int32),
        pltpu.VMEM((CHUNK, D), jnp.float32),
        pltpu.VMEM((CHUNK, D), jnp.float32),
        pltpu.SemaphoreType.DMA,
    ],
)
def _emb_concat(x_hbm, y_hbm, emb_hbm, out_hbm, idx_v, xbuf, ebuf, sem):
    wid = lax.axis_index("s") * NC + lax.axis_index("c")
    for j in range(CHUNKS_PER_W):
        r = wid * CHUNKS_PER_W + j
        pltpu.sync_copy(y_hbm.at[r], idx_v)
        pltpu.async_copy(emb_hbm.at[idx_v], ebuf, sem).wait()
        pltpu.sync_copy(x_hbm.at[pl.ds(r * CHUNK, CHUNK)], xbuf)
        pltpu.sync_copy(xbuf, out_hbm.at[pl.ds(r * CHUNK, CHUNK), pl.ds(0, D)])
        pltpu.sync_copy(ebuf, out_hbm.at[pl.ds(r * CHUNK, CHUNK), pl.ds(D, D)])


def kernel(x, y, embedding):
    y2d = y.astype(jnp.int32).reshape(NIDX_ROWS, CHUNK)
    return _emb_concat(x, y2d, embedding)
